# baseline (speedup 1.0000x reference)
import functools
import math

import jax
import jax.numpy as jnp
from jax import lax
from jax.experimental import pallas as pl
from jax.experimental.pallas import tpu as pltpu
from jax.experimental.pallas import tpu_sc as plsc

_VOCAB = 100000
_DIM = 128
_MAX_LEN = 512
_BATCH = 1024
_SCALE = math.sqrt(float(_DIM))

_NC = 2
_NS = 16
_L = 16
_NW = _NC * _NS
_TOTAL = _BATCH * _MAX_LEN
_ROWS_PER_W = _TOTAL // _NW
_CHUNK = 64
_NCHUNK = _ROWS_PER_W // _CHUNK
_NBUF = 4


@functools.partial(
    pl.kernel,
    mesh=plsc.VectorSubcoreMesh(core_axis_name="c", subcore_axis_name="s"),
    out_type=jax.ShapeDtypeStruct((_TOTAL, _DIM), jnp.float32),
    scratch_types=[
        pltpu.VMEM((_MAX_LEN, _DIM), jnp.float32),
        pltpu.VMEM((_ROWS_PER_W,), jnp.int32),
    ] + [pltpu.VMEM((_CHUNK, _DIM), jnp.float32)] * _NBUF
      + [pltpu.SemaphoreType.DMA] * (2 * _NBUF),
)
def _embed(idx_hbm, table_hbm, pos_hbm, out_hbm, pos_v, idx_v, *bufs_sems):
    rows = bufs_sems[:_NBUF]
    gs = bufs_sems[_NBUF:2 * _NBUF]
    os_ = bufs_sems[2 * _NBUF:]
    wid = lax.axis_index("s") * _NC + lax.axis_index("c")
    base = wid * _ROWS_PER_W
    idx_load = pltpu.make_async_copy(
        idx_hbm.at[pl.ds(base, _ROWS_PER_W)], idx_v, os_[1])
    pos_load = pltpu.make_async_copy(pos_hbm, pos_v, os_[0])
    idx_load.start()
    pos_load.start()

    def gather(c, b):
        return pltpu.make_async_copy(
            table_hbm.at[idx_v.at[pl.ds(c * _CHUNK, _CHUNK)]], rows[b], gs[b])

    def out_copy(c, b):
        return pltpu.make_async_copy(
            rows[b], out_hbm.at[pl.ds(base + c * _CHUNK, _CHUNK)], os_[b])

    _H = _CHUNK // 2

    def out_half(c, b, h):
        return pltpu.make_async_copy(
            rows[b].at[pl.ds(h * _H, _H)],
            out_hbm.at[pl.ds(base + c * _CHUNK + h * _H, _H)], os_[b])

    def compute_half(c, b, h):
        p0 = lax.rem(c * _CHUNK, _MAX_LEN)
        buf = rows[b]

        @plsc.parallel_loop(h * _H, (h + 1) * _H, 1, unroll=4)
        def _(r):
            for j in range(_DIM // _L):
                sl = pl.ds(j * _L, _L)
                buf[r, sl] = buf[r, sl] * _SCALE + pos_v[p0 + r, sl]

    idx_load.wait()
    gather(0, 0).start()
    gather(1, 1).start()
    pos_load.wait()

    def group_body(g, carry):
        for b in range(_NBUF):
            c = _NBUF * g + b
            gather(c, b).wait()
            compute_half(c, b, 0)
            out_half(c, b, 0).start()
            compute_half(c, b, 1)
            out_half(c, b, 1).start()

            @pl.when(c + 2 < _NCHUNK)
            def _():
                b2 = (b + 2) % _NBUF

                @pl.when(c >= 2)
                def _():
                    out_copy(c - 2, b2).wait()

                gather(c + 2, b2).start()

        return carry

    lax.fori_loop(0, _NCHUNK // _NBUF, group_body, 0)
    for k in range(_NBUF):
        c = _NCHUNK - _NBUF + k
        out_copy(c, c % _NBUF).wait()


def kernel(inputs, table, pos_encoding):
    inputs = inputs[:, :_MAX_LEN]
    idx = inputs.reshape(-1)
    out = _embed(idx, table, pos_encoding)
    return out.reshape(inputs.shape[0], inputs.shape[1], _DIM)


# speedup vs baseline: 1.3518x; 1.3518x over previous
import functools
import math

import jax
import jax.numpy as jnp
from jax import lax
from jax.experimental import pallas as pl
from jax.experimental.pallas import tpu as pltpu
from jax.experimental.pallas import tpu_sc as plsc

_VOCAB = 100000
_DIM = 128
_MAX_LEN = 512
_BATCH = 1024
_SCALE = math.sqrt(float(_DIM))

_NC = 2
_NS = 16
_L = 16
_NW = _NC * _NS
_TOTAL = _BATCH * _MAX_LEN
_CHUNK = 128
_NCHUNK = _TOTAL // (_NW * _CHUNK)
_QUART = _MAX_LEN // 4
_NBUF = 4


@functools.partial(
    pl.kernel,
    mesh=plsc.VectorSubcoreMesh(core_axis_name="c", subcore_axis_name="s"),
    out_type=jax.ShapeDtypeStruct((_TOTAL, _DIM), jnp.float32),
    scratch_types=[
        pltpu.VMEM((_QUART, _DIM), jnp.float32),
    ] + [pltpu.VMEM((_CHUNK, _DIM), jnp.float32)] * _NBUF
      + [pltpu.VMEM((_CHUNK,), jnp.int32)] * _NBUF
      + [pltpu.SemaphoreType.DMA] * (3 * _NBUF),
)
def _embed(idx_hbm, table_hbm, pos_hbm, out_hbm, pos_v, *bufs_sems):
    rows = bufs_sems[:_NBUF]
    ib = bufs_sems[_NBUF:2 * _NBUF]
    gs = bufs_sems[2 * _NBUF:3 * _NBUF]
    os_ = bufs_sems[3 * _NBUF:4 * _NBUF]
    is_ = bufs_sems[4 * _NBUF:]
    wid = lax.axis_index("s") * _NC + lax.axis_index("c")
    m = lax.rem(wid, 4)
    base0 = _CHUNK * m + 4 * _CHUNK * lax.div(wid, 4)

    def row0(c):
        return base0 + 32 * _CHUNK * c

    def idx_load(c, b):
        return pltpu.make_async_copy(
            idx_hbm.at[pl.ds(row0(c), _CHUNK)], ib[b], is_[b])

    def gather(c, b):
        return pltpu.make_async_copy(
            table_hbm.at[ib[b]], rows[b], gs[b])

    def out_copy(c, b):
        return pltpu.make_async_copy(
            rows[b], out_hbm.at[pl.ds(row0(c), _CHUNK)], os_[b])

    def compute(b):
        buf = rows[b]

        @plsc.parallel_loop(0, _CHUNK, 1, unroll=4)
        def _(r):
            for j in range(_DIM // _L):
                sl = pl.ds(j * _L, _L)
                buf[r, sl] = buf[r, sl] * _SCALE + pos_v[r, sl]

    pos_load = pltpu.make_async_copy(
        pos_hbm.at[pl.ds(m * _QUART, _QUART)], pos_v, os_[0])
    pos_load.start()
    idx_load(0, 0).start()
    idx_load(1, 1).start()
    idx_load(2, 2).start()
    idx_load(0, 0).wait()
    gather(0, 0).start()
    idx_load(1, 1).wait()
    gather(1, 1).start()
    pos_load.wait()

    def group_body(g, carry):
        for b in range(_NBUF):
            c = _NBUF * g + b
            gather(c, b).wait()
            compute(b)
            out_copy(c, b).start()

            @pl.when(c + 3 < _NCHUNK)
            def _():
                b3 = (b + 3) % _NBUF
                idx_load(c + 3, b3).start()

            @pl.when(c + 2 < _NCHUNK)
            def _():
                b2 = (b + 2) % _NBUF

                @pl.when(c >= 2)
                def _():
                    out_copy(c - 2, b2).wait()

                idx_load(c + 2, b2).wait()
                gather(c + 2, b2).start()

        return carry

    lax.fori_loop(0, _NCHUNK // _NBUF, group_body, 0)
    for k in range(_NBUF):
        c = _NCHUNK - _NBUF + k
        out_copy(c, c % _NBUF).wait()


def kernel(inputs, table, pos_encoding):
    inputs = inputs[:, :_MAX_LEN]
    idx = inputs.reshape(-1)
    out = _embed(idx, table, pos_encoding)
    return out.reshape(inputs.shape[0], inputs.shape[1], _DIM)


# speedup vs baseline: 1.3555x; 1.0027x over previous
import functools
import math

import jax
import jax.numpy as jnp
from jax import lax
from jax.experimental import pallas as pl
from jax.experimental.pallas import tpu as pltpu
from jax.experimental.pallas import tpu_sc as plsc

_VOCAB = 100000
_DIM = 128
_MAX_LEN = 512
_BATCH = 1024
_SCALE = math.sqrt(float(_DIM))

_NC = 2
_NS = 16
_L = 16
_NW = _NC * _NS
_TOTAL = _BATCH * _MAX_LEN
_CHUNK = 128
_NCHUNK = _TOTAL // (_NW * _CHUNK)
_QUART = _MAX_LEN // 4
_NBUF = 4


@functools.partial(
    pl.kernel,
    mesh=plsc.VectorSubcoreMesh(core_axis_name="c", subcore_axis_name="s"),
    out_type=jax.ShapeDtypeStruct((_TOTAL, _DIM), jnp.float32),
    scratch_types=[
        pltpu.VMEM((_QUART, _DIM), jnp.float32),
    ] + [pltpu.VMEM((_CHUNK, _DIM), jnp.float32)] * _NBUF
      + [pltpu.VMEM((_CHUNK,), jnp.int32)] * _NBUF
      + [pltpu.SemaphoreType.DMA] * (3 * _NBUF),
)
def _embed(idx_hbm, table_hbm, pos_hbm, out_hbm, pos_v, *bufs_sems):
    rows = bufs_sems[:_NBUF]
    ib = bufs_sems[_NBUF:2 * _NBUF]
    gs = bufs_sems[2 * _NBUF:3 * _NBUF]
    os_ = bufs_sems[3 * _NBUF:4 * _NBUF]
    is_ = bufs_sems[4 * _NBUF:]
    wid = lax.axis_index("s") * _NC + lax.axis_index("c")
    m = lax.rem(wid, 4)
    base0 = _CHUNK * m + 4 * _CHUNK * lax.div(wid, 4)

    def row0(c):
        return base0 + 32 * _CHUNK * c

    def idx_load(c, b):
        return pltpu.make_async_copy(
            idx_hbm.at[pl.ds(row0(c), _CHUNK)], ib[b], is_[b])

    def gather(c, b):
        return pltpu.make_async_copy(
            table_hbm.at[ib[b]], rows[b], gs[b])

    def out_copy(c, b):
        return pltpu.make_async_copy(
            rows[b], out_hbm.at[pl.ds(row0(c), _CHUNK)], os_[b])

    def compute(b):
        buf = rows[b]

        @plsc.parallel_loop(0, _CHUNK, 1, unroll=8)
        def _(r):
            for j in range(_DIM // _L):
                sl = pl.ds(j * _L, _L)
                buf[r, sl] = buf[r, sl] * _SCALE + pos_v[r, sl]

    pos_load = pltpu.make_async_copy(
        pos_hbm.at[pl.ds(m * _QUART, _QUART)], pos_v, os_[0])
    pos_load.start()
    idx_load(0, 0).start()
    idx_load(1, 1).start()
    idx_load(2, 2).start()
    idx_load(0, 0).wait()
    gather(0, 0).start()
    idx_load(1, 1).wait()
    gather(1, 1).start()
    pos_load.wait()

    def group_body(g, carry):
        for b in range(_NBUF):
            c = _NBUF * g + b
            gather(c, b).wait()
            compute(b)
            out_copy(c, b).start()

            @pl.when(c + 3 < _NCHUNK)
            def _():
                b3 = (b + 3) % _NBUF
                idx_load(c + 3, b3).start()

            @pl.when(c + 2 < _NCHUNK)
            def _():
                b2 = (b + 2) % _NBUF

                @pl.when(c >= 2)
                def _():
                    out_copy(c - 2, b2).wait()

                idx_load(c + 2, b2).wait()
                gather(c + 2, b2).start()

        return carry

    lax.fori_loop(0, _NCHUNK // _NBUF, group_body, 0)
    for k in range(_NBUF):
        c = _NCHUNK - _NBUF + k
        out_copy(c, c % _NBUF).wait()


def kernel(inputs, table, pos_encoding):
    inputs = inputs[:, :_MAX_LEN]
    idx = inputs.reshape(-1)
    out = _embed(idx, table, pos_encoding)
    return out.reshape(inputs.shape[0], inputs.shape[1], _DIM)
